# baseline (speedup 1.0000x reference)
import functools

import jax
import jax.numpy as jnp
from jax import lax
from jax.experimental import pallas as pl
from jax.experimental.pallas import tpu as pltpu
from jax.experimental.pallas import tpu_sc as plsc

N = 20000
G = 128
NC = 2
NS = 16
L = 16
NW = NC * NS
CHUNK = 640
NPAD = NW * CHUNK
NGROUP = CHUNK // L
GL = G * L
UNROLL = 8
POS_THR = 0.5
NEG_THR = 0.4

_mesh = plsc.VectorSubcoreMesh(core_axis_name="c", subcore_axis_name="s")

_GDN = lax.GatherDimensionNumbers(
    offset_dims=(), collapsed_slice_dims=(0,), start_index_map=(0,))


def _shuf(x, idx):
    return lax.gather(x, idx[:, None], _GDN, (1,),
                      mode=lax.GatherScatterMode.PROMISE_IN_BOUNDS)


_f32 = jnp.float32
_i32 = jnp.int32


@functools.partial(
    pl.kernel,
    out_type=[
        jax.ShapeDtypeStruct((NPAD,), _f32),
        jax.ShapeDtypeStruct((NPAD,), _i32),
        jax.ShapeDtypeStruct((NW * G,), _f32),
        jax.ShapeDtypeStruct((NW, NGROUP, GL), _f32),
    ],
    mesh=_mesh,
    scratch_types=[
        pltpu.VMEM((CHUNK,), _f32),
        pltpu.VMEM((CHUNK,), _f32),
        pltpu.VMEM((CHUNK,), _f32),
        pltpu.VMEM((CHUNK,), _f32),
        pltpu.VMEM((GL,), _f32),
        pltpu.VMEM((GL,), _f32),
        pltpu.VMEM((GL,), _f32),
        pltpu.VMEM((GL,), _f32),
        pltpu.VMEM((GL,), _f32),
        pltpu.VMEM((GL,), _f32),
        pltpu.VMEM((GL,), _f32),
        pltpu.VMEM((GL,), _f32),
        pltpu.VMEM((CHUNK,), _f32),
        pltpu.VMEM((CHUNK,), _i32),
        pltpu.VMEM((G,), _f32),
        pltpu.SemaphoreType.DMA,
        pltpu.SemaphoreType.DMA,
    ],
)
def _phase1(ax1_h, ay1_h, ax2_h, ay2_h, gx1_h, gy1_h, gx2_h, gy2_h,
            rowmax_h, argmax_h, colpart_h, ovmat_h,
            sax1, say1, sax2, say2, sgx1, sgy1, sgx2, sgy2, sga,
            sovA, sovB, scol, srm, sam, scolred, semA, semB):
    cid = lax.axis_index("c")
    sid = lax.axis_index("s")
    w = sid * NC + cid
    base = w * CHUNK

    pltpu.sync_copy(ax1_h.at[pl.ds(base, CHUNK)], sax1)
    pltpu.sync_copy(ay1_h.at[pl.ds(base, CHUNK)], say1)
    pltpu.sync_copy(ax2_h.at[pl.ds(base, CHUNK)], sax2)
    pltpu.sync_copy(ay2_h.at[pl.ds(base, CHUNK)], say2)
    pltpu.sync_copy(gx1_h, sgx1)
    pltpu.sync_copy(gy1_h, sgy1)
    pltpu.sync_copy(gx2_h, sgx2)
    pltpu.sync_copy(gy2_h, sgy2)

    def init_j(j, _):
        jb = j * L
        sga[pl.ds(jb, L)] = (
            (sgx2[pl.ds(jb, L)] - sgx1[pl.ds(jb, L)]) *
            (sgy2[pl.ds(jb, L)] - sgy1[pl.ds(jb, L)]))
        scol[pl.ds(jb, L)] = jnp.full((L,), -1.0, _f32)
        return 0
    lax.fori_loop(0, G, init_j, 0)

    def one_group(g, sov):
        gb = g * L
        av1 = sax1[pl.ds(gb, L)]
        au1 = say1[pl.ds(gb, L)]
        av2 = sax2[pl.ds(gb, L)]
        au2 = say2[pl.ds(gb, L)]
        aarea = (av2 - av1) * (au2 - au1)

        def iou_at(j):
            jb = j * L
            ltx = jnp.maximum(av1, sgx1[pl.ds(jb, L)])
            lty = jnp.maximum(au1, sgy1[pl.ds(jb, L)])
            rbx = jnp.minimum(av2, sgx2[pl.ds(jb, L)])
            rby = jnp.minimum(au2, sgy2[pl.ds(jb, L)])
            iw = jnp.maximum(rbx - ltx, 0.0)
            ih = jnp.maximum(rby - lty, 0.0)
            inter = iw * ih
            union = jnp.maximum(aarea + sga[pl.ds(jb, L)] - inter, 1e-9)
            return inter / union

        def j_body(u, carry):
            rm, am = carry
            j0 = u * UNROLL
            ious = []
            for k in range(UNROLL):
                j = j0 + k
                jb = j * L
                iou = iou_at(j)
                sov[pl.ds(jb, L)] = iou
                scol[pl.ds(jb, L)] = jnp.maximum(scol[pl.ds(jb, L)], iou)
                ious.append(iou)
            ms = list(ious)
            as_ = [j0 + k for k in range(UNROLL)]
            while len(ms) > 1:
                nm, na = [], []
                for p in range(0, len(ms), 2):
                    nm.append(jnp.maximum(ms[p], ms[p + 1]))
                    na.append(jnp.where(ms[p + 1] > ms[p],
                                        as_[p + 1], as_[p]))
                ms, as_ = nm, na
            am = jnp.where(ms[0] > rm, as_[0], am)
            rm = jnp.maximum(rm, ms[0])
            return rm, am

        rm0 = jnp.full((L,), -1.0, _f32)
        am0 = jnp.zeros((L,), _i32)
        rm, am = lax.fori_loop(0, G // UNROLL, j_body, (rm0, am0))
        srm[pl.ds(gb, L)] = rm
        sam[pl.ds(gb, L)] = am

    def dgroup(t, _):
        for parity, sov, sem in ((0, sovA, semA), (1, sovB, semB)):
            g = t * 2 + parity

            @pl.when(t > 0)
            def _wait():
                pltpu.make_async_copy(sov, ovmat_h.at[w, g - 2], sem).wait()

            one_group(g, sov)
            pltpu.make_async_copy(sov, ovmat_h.at[w, g], sem).start()
        return 0
    lax.fori_loop(0, NGROUP // 2, dgroup, 0)
    pltpu.make_async_copy(sovA, ovmat_h.at[w, NGROUP - 2], semA).wait()
    pltpu.make_async_copy(sovB, ovmat_h.at[w, NGROUP - 1], semB).wait()

    lane = lax.iota(_i32, L)

    def red_outer(jv, _):
        def red_inner(jl, acc):
            m = scol[pl.ds((jv * L + jl) * L, L)]
            for sh in (8, 4, 2, 1):
                m = jnp.maximum(m, _shuf(m, lane ^ sh))
            return jnp.where(lane == jl, m, acc)
        acc = lax.fori_loop(0, L, red_inner, jnp.full((L,), -1.0, _f32))
        scolred[pl.ds(jv * L, L)] = acc
        return 0
    lax.fori_loop(0, G // L, red_outer, 0)

    pltpu.sync_copy(srm, rowmax_h.at[pl.ds(base, CHUNK)])
    pltpu.sync_copy(sam, argmax_h.at[pl.ds(base, CHUNK)])
    pltpu.sync_copy(scolred, colpart_h.at[pl.ds(w * G, G)])


@functools.partial(
    pl.kernel,
    out_type=[
        jax.ShapeDtypeStruct((NPAD,), _i32),
        jax.ShapeDtypeStruct((NPAD,), _i32),
    ],
    mesh=_mesh,
    scratch_types=[
        pltpu.VMEM((GL,), _f32),
        pltpu.VMEM((GL,), _f32),
        pltpu.VMEM((CHUNK,), _f32),
        pltpu.VMEM((CHUNK,), _i32),
        pltpu.VMEM((NW * G,), _f32),
        pltpu.VMEM((GL,), _f32),
        pltpu.VMEM((G,), _i32),
        pltpu.VMEM((CHUNK,), _i32),
        pltpu.VMEM((CHUNK,), _i32),
        pltpu.SemaphoreType.DMA,
        pltpu.SemaphoreType.DMA,
    ],
)
def _phase2(rowmax_h, argmax_h, colpart_h, ovmat_h, glab_h,
            assigned_h, labels_h,
            sovA, sovB, srm, sam, scp, scolb, slab, sasg, slabo,
            semA, semB):
    cid = lax.axis_index("c")
    sid = lax.axis_index("s")
    w = sid * NC + cid
    base = w * CHUNK

    pltpu.make_async_copy(ovmat_h.at[w, 0], sovA, semA).start()
    pltpu.make_async_copy(ovmat_h.at[w, 1], sovB, semB).start()
    pltpu.sync_copy(rowmax_h.at[pl.ds(base, CHUNK)], srm)
    pltpu.sync_copy(argmax_h.at[pl.ds(base, CHUNK)], sam)
    pltpu.sync_copy(colpart_h, scp)
    pltpu.sync_copy(glab_h, slab)

    lane = lax.iota(_i32, L)
    for jv in range(G // L):
        acc = scp[pl.ds(jv * L, L)]
        for wi in range(1, NW):
            acc = jnp.maximum(acc, scp[pl.ds(wi * G + jv * L, L)])
        for jl in range(L):
            scolb[pl.ds((jv * L + jl) * L, L)] = _shuf(
                acc, jnp.full((L,), jl, _i32))

    labv = [slab[pl.ds(v * L, L)] for v in range(G // L)]

    def one_group(g, sov):
        gb = g * L
        rm = srm[pl.ds(gb, L)]
        am = sam[pl.ds(gb, L)]
        asg = jnp.where(rm > POS_THR, am + 1,
                        jnp.where(rm < NEG_THR, 0, -1))

        def j_body(u, lastj):
            j0 = u * UNROLL
            c = jnp.full((L,), -1, _i32)
            for k in range(UNROLL):
                j = j0 + k
                jb = j * L
                match = sov[pl.ds(jb, L)] == scolb[pl.ds(jb, L)]
                c = jnp.where(match, j, c)
            return jnp.where(c >= 0, c, lastj)
        lastj = lax.fori_loop(0, G // UNROLL, j_body, jnp.full((L,), -1, _i32))

        asg = jnp.where(lastj >= 0, lastj + 1, asg)
        safe = jnp.clip(asg - 1, 0, G - 1)
        lo = safe & (L - 1)
        hi = safe >> 4
        lbl = _shuf(labv[0], lo)
        for v in range(1, G // L):
            lbl = jnp.where(hi == v, _shuf(labv[v], lo), lbl)
        sasg[pl.ds(gb, L)] = asg
        slabo[pl.ds(gb, L)] = jnp.where(asg > 0, lbl, -1)

    def dgroup(t, _):
        for parity, sov, sem in ((0, sovA, semA), (1, sovB, semB)):
            g = t * 2 + parity
            pltpu.make_async_copy(ovmat_h.at[w, g], sov, sem).wait()
            one_group(g, sov)

            @pl.when(t < NGROUP // 2 - 1)
            def _prefetch():
                pltpu.make_async_copy(ovmat_h.at[w, g + 2], sov, sem).start()
        return 0
    lax.fori_loop(0, NGROUP // 2, dgroup, 0)

    pltpu.sync_copy(sasg, assigned_h.at[pl.ds(base, CHUNK)])
    pltpu.sync_copy(slabo, labels_h.at[pl.ds(base, CHUNK)])


def kernel(bboxes, targets, num_level_bboxes):
    del num_level_bboxes
    bb = jnp.pad(bboxes, ((0, NPAD - N), (0, 0)))
    ax1, ay1, ax2, ay2 = (bb[:, k] for k in range(4))
    gx1b, gy1b, gx2b, gy2b = (
        jnp.broadcast_to(targets[:, k:k + 1], (G, L)).reshape(GL)
        for k in range(4))
    glab = targets[:, 4].astype(_i32)

    rowmax, argmax, colpart, ovmat = _phase1(
        ax1, ay1, ax2, ay2, gx1b, gy1b, gx2b, gy2b)
    assigned, labels = _phase2(rowmax, argmax, colpart, ovmat, glab)
    return assigned[:N], rowmax[:N], labels[:N]


# speedup vs baseline: 2.1610x; 2.1610x over previous
import functools

import jax
import jax.numpy as jnp
from jax import lax
from jax.experimental import pallas as pl
from jax.experimental.pallas import tpu as pltpu
from jax.experimental.pallas import tpu_sc as plsc

N = 20000
G = 128
NC = 2
NS = 16
L = 16
NW = NC * NS
CHUNK = 640
NPAD = NW * CHUNK
NGROUP = CHUNK // L
GL = G * L
UNROLL = 4
POS_THR = 0.5
NEG_THR = 0.4

_mesh = plsc.VectorSubcoreMesh(core_axis_name="c", subcore_axis_name="s")

_GDN = lax.GatherDimensionNumbers(
    offset_dims=(), collapsed_slice_dims=(0,), start_index_map=(0,))


def _shuf(x, idx):
    return lax.gather(x, idx[:, None], _GDN, (1,),
                      mode=lax.GatherScatterMode.PROMISE_IN_BOUNDS)


_f32 = jnp.float32
_i32 = jnp.int32


@functools.partial(
    pl.kernel,
    out_type=[
        jax.ShapeDtypeStruct((NPAD,), _f32),
        jax.ShapeDtypeStruct((NPAD,), _i32),
        jax.ShapeDtypeStruct((NW * G,), _f32),
        jax.ShapeDtypeStruct((NW, NGROUP, GL), _f32),
    ],
    mesh=_mesh,
    scratch_types=[
        pltpu.VMEM((CHUNK,), _f32),
        pltpu.VMEM((CHUNK,), _f32),
        pltpu.VMEM((CHUNK,), _f32),
        pltpu.VMEM((CHUNK,), _f32),
        pltpu.VMEM((GL,), _f32),
        pltpu.VMEM((GL,), _f32),
        pltpu.VMEM((GL,), _f32),
        pltpu.VMEM((GL,), _f32),
        pltpu.VMEM((GL,), _f32),
        pltpu.VMEM((GL,), _f32),
        pltpu.VMEM((GL,), _f32),
        pltpu.VMEM((GL,), _f32),
        pltpu.VMEM((CHUNK,), _f32),
        pltpu.VMEM((CHUNK,), _i32),
        pltpu.VMEM((G,), _f32),
        pltpu.SemaphoreType.DMA,
        pltpu.SemaphoreType.DMA,
    ],
)
def _phase1(ax1_h, ay1_h, ax2_h, ay2_h, gx1_h, gy1_h, gx2_h, gy2_h,
            rowmax_h, argmax_h, colpart_h, ovmat_h,
            sax1, say1, sax2, say2, sgx1, sgy1, sgx2, sgy2, sga,
            sovA, sovB, scol, srm, sam, scolred, semA, semB):
    cid = lax.axis_index("c")
    sid = lax.axis_index("s")
    w = sid * NC + cid
    base = w * CHUNK

    pltpu.sync_copy(ax1_h.at[pl.ds(base, CHUNK)], sax1)
    pltpu.sync_copy(ay1_h.at[pl.ds(base, CHUNK)], say1)
    pltpu.sync_copy(ax2_h.at[pl.ds(base, CHUNK)], sax2)
    pltpu.sync_copy(ay2_h.at[pl.ds(base, CHUNK)], say2)
    pltpu.sync_copy(gx1_h, sgx1)
    pltpu.sync_copy(gy1_h, sgy1)
    pltpu.sync_copy(gx2_h, sgx2)
    pltpu.sync_copy(gy2_h, sgy2)

    def init_j(j, _):
        jb = j * L
        sga[pl.ds(jb, L)] = (
            (sgx2[pl.ds(jb, L)] - sgx1[pl.ds(jb, L)]) *
            (sgy2[pl.ds(jb, L)] - sgy1[pl.ds(jb, L)]))
        scol[pl.ds(jb, L)] = jnp.full((L,), -1.0, _f32)
        return 0
    lax.fori_loop(0, G, init_j, 0)

    def one_group(g, sov):
        gb = g * L
        av1 = sax1[pl.ds(gb, L)]
        au1 = say1[pl.ds(gb, L)]
        av2 = sax2[pl.ds(gb, L)]
        au2 = say2[pl.ds(gb, L)]
        aarea = (av2 - av1) * (au2 - au1)

        def iou_at(j):
            jb = j * L
            ltx = jnp.maximum(av1, sgx1[pl.ds(jb, L)])
            lty = jnp.maximum(au1, sgy1[pl.ds(jb, L)])
            rbx = jnp.minimum(av2, sgx2[pl.ds(jb, L)])
            rby = jnp.minimum(au2, sgy2[pl.ds(jb, L)])
            iw = jnp.maximum(rbx - ltx, 0.0)
            ih = jnp.maximum(rby - lty, 0.0)
            inter = iw * ih
            union = jnp.maximum(aarea + sga[pl.ds(jb, L)] - inter, 1e-9)
            return inter / union

        def j_body(u, carry):
            rm, am = carry
            j0 = u * UNROLL
            ious = []
            for k in range(UNROLL):
                j = j0 + k
                jb = j * L
                iou = iou_at(j)
                sov[pl.ds(jb, L)] = iou
                scol[pl.ds(jb, L)] = jnp.maximum(scol[pl.ds(jb, L)], iou)
                ious.append(iou)
            ms = list(ious)
            as_ = [j0 + k for k in range(UNROLL)]
            while len(ms) > 1:
                nm, na = [], []
                for p in range(0, len(ms), 2):
                    nm.append(jnp.maximum(ms[p], ms[p + 1]))
                    na.append(jnp.where(ms[p + 1] > ms[p],
                                        as_[p + 1], as_[p]))
                ms, as_ = nm, na
            am = jnp.where(ms[0] > rm, as_[0], am)
            rm = jnp.maximum(rm, ms[0])
            return rm, am

        rm0 = jnp.full((L,), -1.0, _f32)
        am0 = jnp.zeros((L,), _i32)
        rm, am = lax.fori_loop(0, G // UNROLL, j_body, (rm0, am0))
        srm[pl.ds(gb, L)] = rm
        sam[pl.ds(gb, L)] = am

    def dgroup(t, _):
        for parity, sov, sem in ((0, sovA, semA), (1, sovB, semB)):
            g = t * 2 + parity

            @pl.when(t > 0)
            def _wait():
                pltpu.make_async_copy(sov, ovmat_h.at[w, g - 2], sem).wait()

            one_group(g, sov)
            pltpu.make_async_copy(sov, ovmat_h.at[w, g], sem).start()
        return 0
    lax.fori_loop(0, NGROUP // 2, dgroup, 0)
    pltpu.make_async_copy(sovA, ovmat_h.at[w, NGROUP - 2], semA).wait()
    pltpu.make_async_copy(sovB, ovmat_h.at[w, NGROUP - 1], semB).wait()

    lane = lax.iota(_i32, L)

    def red_outer(jv, _):
        def red_inner(jl, acc):
            m = scol[pl.ds((jv * L + jl) * L, L)]
            for sh in (8, 4, 2, 1):
                m = jnp.maximum(m, _shuf(m, lane ^ sh))
            return jnp.where(lane == jl, m, acc)
        acc = lax.fori_loop(0, L, red_inner, jnp.full((L,), -1.0, _f32))
        scolred[pl.ds(jv * L, L)] = acc
        return 0
    lax.fori_loop(0, G // L, red_outer, 0)

    pltpu.sync_copy(srm, rowmax_h.at[pl.ds(base, CHUNK)])
    pltpu.sync_copy(sam, argmax_h.at[pl.ds(base, CHUNK)])
    pltpu.sync_copy(scolred, colpart_h.at[pl.ds(w * G, G)])


@functools.partial(
    pl.kernel,
    out_type=[
        jax.ShapeDtypeStruct((NPAD,), _i32),
        jax.ShapeDtypeStruct((NPAD,), _i32),
    ],
    mesh=_mesh,
    scratch_types=[
        pltpu.VMEM((GL,), _f32),
        pltpu.VMEM((GL,), _f32),
        pltpu.VMEM((CHUNK,), _f32),
        pltpu.VMEM((CHUNK,), _i32),
        pltpu.VMEM((NW * G,), _f32),
        pltpu.VMEM((GL,), _f32),
        pltpu.VMEM((G,), _i32),
        pltpu.VMEM((CHUNK,), _i32),
        pltpu.VMEM((CHUNK,), _i32),
        pltpu.SemaphoreType.DMA,
        pltpu.SemaphoreType.DMA,
    ],
)
def _phase2(rowmax_h, argmax_h, colpart_h, ovmat_h, glab_h,
            assigned_h, labels_h,
            sovA, sovB, srm, sam, scp, scolb, slab, sasg, slabo,
            semA, semB):
    cid = lax.axis_index("c")
    sid = lax.axis_index("s")
    w = sid * NC + cid
    base = w * CHUNK

    pltpu.make_async_copy(ovmat_h.at[w, 0], sovA, semA).start()
    pltpu.make_async_copy(ovmat_h.at[w, 1], sovB, semB).start()
    pltpu.sync_copy(rowmax_h.at[pl.ds(base, CHUNK)], srm)
    pltpu.sync_copy(argmax_h.at[pl.ds(base, CHUNK)], sam)
    pltpu.sync_copy(colpart_h, scp)
    pltpu.sync_copy(glab_h, slab)

    lane = lax.iota(_i32, L)
    for jv in range(G // L):
        acc = scp[pl.ds(jv * L, L)]
        for wi in range(1, NW):
            acc = jnp.maximum(acc, scp[pl.ds(wi * G + jv * L, L)])
        for jl in range(L):
            scolb[pl.ds((jv * L + jl) * L, L)] = _shuf(
                acc, jnp.full((L,), jl, _i32))

    labv = [slab[pl.ds(v * L, L)] for v in range(G // L)]

    def one_group(g, sov):
        gb = g * L
        rm = srm[pl.ds(gb, L)]
        am = sam[pl.ds(gb, L)]
        asg = jnp.where(rm > POS_THR, am + 1,
                        jnp.where(rm < NEG_THR, 0, -1))

        def j_body(u, lastj):
            j0 = u * UNROLL
            c = jnp.full((L,), -1, _i32)
            for k in range(UNROLL):
                j = j0 + k
                jb = j * L
                match = sov[pl.ds(jb, L)] == scolb[pl.ds(jb, L)]
                c = jnp.where(match, j, c)
            return jnp.where(c >= 0, c, lastj)
        lastj = lax.fori_loop(0, G // UNROLL, j_body, jnp.full((L,), -1, _i32))

        asg = jnp.where(lastj >= 0, lastj + 1, asg)
        safe = jnp.clip(asg - 1, 0, G - 1)
        lo = safe & (L - 1)
        hi = safe >> 4
        lbl = _shuf(labv[0], lo)
        for v in range(1, G // L):
            lbl = jnp.where(hi == v, _shuf(labv[v], lo), lbl)
        sasg[pl.ds(gb, L)] = asg
        slabo[pl.ds(gb, L)] = jnp.where(asg > 0, lbl, -1)

    def dgroup(t, _):
        for parity, sov, sem in ((0, sovA, semA), (1, sovB, semB)):
            g = t * 2 + parity
            pltpu.make_async_copy(ovmat_h.at[w, g], sov, sem).wait()
            one_group(g, sov)

            @pl.when(t < NGROUP // 2 - 1)
            def _prefetch():
                pltpu.make_async_copy(ovmat_h.at[w, g + 2], sov, sem).start()
        return 0
    lax.fori_loop(0, NGROUP // 2, dgroup, 0)

    pltpu.sync_copy(sasg, assigned_h.at[pl.ds(base, CHUNK)])
    pltpu.sync_copy(slabo, labels_h.at[pl.ds(base, CHUNK)])


def kernel(bboxes, targets, num_level_bboxes):
    del num_level_bboxes
    bb = jnp.pad(bboxes, ((0, NPAD - N), (0, 0)))
    ax1, ay1, ax2, ay2 = (bb[:, k] for k in range(4))
    gx1b, gy1b, gx2b, gy2b = (
        jnp.broadcast_to(targets[:, k:k + 1], (G, L)).reshape(GL)
        for k in range(4))
    glab = targets[:, 4].astype(_i32)

    rowmax, argmax, colpart, ovmat = _phase1(
        ax1, ay1, ax2, ay2, gx1b, gy1b, gx2b, gy2b)
    assigned, labels = _phase2(rowmax, argmax, colpart, ovmat, glab)
    return assigned[:N], rowmax[:N], labels[:N]
